# baseline (speedup 1.0000x reference)
import functools

import jax
import jax.numpy as jnp
import numpy as np
from jax import lax
from jax.experimental import pallas as pl
from jax.experimental.pallas import tpu as pltpu
from jax.experimental.pallas import tpu_sc as plsc

_N_EDGE = 1600000
_D_E = 16
_N_NODE = 50000
_GRID = 25
_CW = _N_EDGE // _GRID

_N_WORK = 32
_ROWS_PER_SUB = 1560
_SC_ROWS = _N_WORK * _ROWS_PER_SUB
_TAIL_ROWS = _N_NODE - _SC_ROWS
_NCH = 5
_CHROWS = _ROWS_PER_SUB // _NCH


def _edge_body(g_ref, e_ref, w_ref, b_ref, o_ref, acc_e):
    step = pl.program_id(0)

    @pl.when(step == 0)
    def _init():
        acc_e[...] = jnp.zeros_like(acc_e)

    acc_e[...] += e_ref[...]

    @pl.when(step == pl.num_programs(0) - 1)
    def _finish():
        e_sum = jnp.sum(acc_e[...], axis=1, keepdims=True)
        out = jax.lax.dot_general(
            g_ref[...], w_ref[0:128, :], (((1,), (0,)), ((), ())),
            preferred_element_type=jnp.float32,
        )
        out += jax.lax.dot_general(
            e_sum * (1.0 / _N_EDGE), w_ref[128:144, :],
            (((0,), (0,)), ((), ())),
            preferred_element_type=jnp.float32,
        )
        o_ref[...] = out + b_ref[...]


def _combine_body(o1_ref, nps_ref, tail_ref, w_ref, o_ref):
    n_sum = jnp.sum(nps_ref[...], axis=0, keepdims=True)
    n_sum += jnp.sum(tail_ref[...], axis=0, keepdims=True)
    o_ref[...] = o1_ref[...] + jax.lax.dot_general(
        n_sum * (1.0 / _N_NODE), w_ref[144:272, :],
        (((1,), (0,)), ((), ())),
        preferred_element_type=jnp.float32,
    )


def _sc_node_body(n_hbm, out_hbm, buf0, buf1, row_v, sem0, sem1):
    cid = lax.axis_index("c")
    sid = lax.axis_index("s")
    wid = sid * 2 + cid
    base = wid * _ROWS_PER_SUB
    bufs = (buf0, buf1)
    sems = (sem0, sem1)

    def copy(i, b):
        return pltpu.make_async_copy(
            n_hbm.at[pl.ds(base + i * _CHROWS, _CHROWS), :],
            bufs[b], sems[b],
        )

    copy(0, 0).start()
    copy(1, 1).start()

    acc = [jnp.zeros((16,), jnp.float32) for _ in range(8)]
    for i in range(_NCH):
        b = i % 2
        copy(i, b).wait()
        buf = bufs[b]

        def chunk_step(r, carry):
            return tuple(
                carry[j] + buf[r, pl.ds(16 * j, 16)] for j in range(8)
            )

        acc = list(lax.fori_loop(0, _CHROWS, chunk_step, tuple(acc)))
        if i + 2 < _NCH:
            copy(i + 2, b).start()

    for j in range(8):
        row_v[pl.ds(16 * j, 16)] = acc[j]
    pltpu.sync_copy(row_v, out_hbm.at[pl.ds(wid * 128, 128)])


@jax.jit
def kernel(global_attr, edge_attr, node_attr, W, b):
    e_t = edge_attr.T
    g_row = global_attr.reshape(1, 128)
    b_row = b.reshape(1, 128)

    o1 = pl.pallas_call(
        _edge_body,
        grid=(_GRID,),
        in_specs=[
            pl.BlockSpec((1, 128), lambda i: (0, 0)),
            pl.BlockSpec((_D_E, _CW), lambda i: (0, i)),
            pl.BlockSpec((272, 128), lambda i: (0, 0)),
            pl.BlockSpec((1, 128), lambda i: (0, 0)),
        ],
        out_specs=pl.BlockSpec((1, 128), lambda i: (0, 0)),
        out_shape=jax.ShapeDtypeStruct((1, 128), jnp.float32),
        scratch_shapes=[pltpu.VMEM((_D_E, _CW), jnp.float32)],
    )(g_row, e_t, W, b_row)

    sc_node = functools.partial(
        pl.kernel,
        mesh=plsc.VectorSubcoreMesh(core_axis_name="c", subcore_axis_name="s"),
        out_type=jax.ShapeDtypeStruct((_N_WORK * 128,), jnp.float32),
        scratch_types=[
            pltpu.VMEM((_CHROWS, 128), jnp.float32),
            pltpu.VMEM((_CHROWS, 128), jnp.float32),
            pltpu.VMEM((128,), jnp.float32),
            pltpu.SemaphoreType.DMA,
            pltpu.SemaphoreType.DMA,
        ],
    )(_sc_node_body)
    nps = sc_node(node_attr).reshape(_N_WORK, 128)

    out_row = pl.pallas_call(
        _combine_body,
        grid=(1,),
        in_specs=[
            pl.BlockSpec((1, 128), lambda i: (0, 0)),
            pl.BlockSpec((_N_WORK, 128), lambda i: (0, 0)),
            pl.BlockSpec((_TAIL_ROWS, 128), lambda i: (_SC_ROWS // _TAIL_ROWS, 0)),
            pl.BlockSpec((272, 128), lambda i: (0, 0)),
        ],
        out_specs=pl.BlockSpec((1, 128), lambda i: (0, 0)),
        out_shape=jax.ShapeDtypeStruct((1, 128), jnp.float32),
    )(o1, nps, node_attr, W)
    return out_row.reshape(128)


# speedup vs baseline: 1.3291x; 1.3291x over previous
import jax
import jax.numpy as jnp
import numpy as np
from jax.experimental import pallas as pl
from jax.experimental.pallas import tpu as pltpu

_N_EDGE = 1600000
_D_E = 16
_N_NODE = 50000
_ESTEPS = 20
_NSTEPS = 5
_GRID = _ESTEPS + _NSTEPS
_CW = _N_EDGE // _ESTEPS
_NBLK = _N_NODE // _NSTEPS


def _body(g_ref, e_ref, n_ref, w_ref, b_ref, o_ref, acc_e, acc_n):
    step = pl.program_id(0)

    @pl.when(step == 0)
    def _init():
        acc_e[...] = jnp.zeros_like(acc_e)
        acc_n[...] = jnp.zeros_like(acc_n)

    @pl.when(step < _ESTEPS)
    def _edge_phase():
        acc_e[...] += e_ref[...]

    @pl.when(step >= _ESTEPS)
    def _node_phase():
        acc_n[...] += jnp.sum(n_ref[...].reshape(_NBLK // 8, 8, 128), axis=0)

    @pl.when(step == pl.num_programs(0) - 1)
    def _finish():
        e_sum = jnp.sum(acc_e[...], axis=1, keepdims=True)
        dn = (((1,), (0,)), ((), ()))
        out = jax.lax.dot_general(
            g_ref[...], w_ref[0:128, :], dn,
            preferred_element_type=jnp.float32,
        )
        out += jax.lax.dot_general(
            e_sum * (1.0 / _N_EDGE), w_ref[128:144, :],
            (((0,), (0,)), ((), ())),
            preferred_element_type=jnp.float32,
        )
        n_sum = jnp.sum(acc_n[...], axis=0, keepdims=True)
        out += jax.lax.dot_general(
            n_sum * (1.0 / _N_NODE), w_ref[144:272, :], dn,
            preferred_element_type=jnp.float32,
        )
        o_ref[...] = out + b_ref[...]


@jax.jit
def kernel(global_attr, edge_attr, node_attr, W, b):
    e_t = edge_attr.T
    g_row = global_attr.reshape(1, 128)
    b_row = b.reshape(1, 128)

    out_row = pl.pallas_call(
        _body,
        grid=(_GRID,),
        in_specs=[
            pl.BlockSpec((1, 128), lambda i: (0, 0)),
            pl.BlockSpec((_D_E, _CW), lambda i: (0, jnp.minimum(i, _ESTEPS - 1))),
            pl.BlockSpec((_NBLK, 128), lambda i: (jnp.maximum(i - _ESTEPS, 0), 0)),
            pl.BlockSpec((272, 128), lambda i: (0, 0)),
            pl.BlockSpec((1, 128), lambda i: (0, 0)),
        ],
        out_specs=pl.BlockSpec((1, 128), lambda i: (0, 0)),
        out_shape=jax.ShapeDtypeStruct((1, 128), jnp.float32),
        scratch_shapes=[
            pltpu.VMEM((_D_E, _CW), jnp.float32),
            pltpu.VMEM((8, 128), jnp.float32),
        ],
    )(g_row, e_t, node_attr, W, b_row)
    return out_row.reshape(128)
